# Initial kernel scaffold; baseline (speedup 1.0000x reference)
import jax
import jax.numpy as jnp
from jax.experimental import pallas as pl


def kernel(edge_index, entity_embeddings, W1, b1, W2, b2, Wc, bc):
    raise NotImplementedError("write your pallas kernel here")


# speedup vs baseline: 6.1547x; 6.1547x over previous
import functools

import jax
import jax.numpy as jnp
from jax import lax
from jax.experimental import pallas as pl
from jax.experimental.pallas import tpu as pltpu
from jax.experimental.pallas import tpu_sc as plsc

N_NODES = 10000
N_EDGES = 320000
DIM = 128
NUM_CLASSES = 10
EPS = 1e-15

_ROWS_PER_BLK = 1000
_N_BLKS = N_NODES // _ROWS_PER_BLK

_CHUNK = 128
_N_CHUNKS = N_EDGES // _CHUNK
_NW = 32
_FULL = _N_CHUNKS // _NW
_EXTRA = _N_CHUNKS - _FULL * _NW
_STRIPE = 624
_STRIPE_EXTRA = 8


def _logmap0(x):
    sq = jnp.sum(x * x, axis=-1, keepdims=True)
    norm = jnp.maximum(jnp.sqrt(sq), EPS)
    arg = jnp.clip(norm, 0.0, 1.0 - 1e-6)
    att = 0.5 * jnp.log((1.0 + arg) / (1.0 - arg))
    return x * (att / norm)


def _expmap0(u):
    sq = jnp.sum(u * u, axis=-1, keepdims=True)
    norm = jnp.maximum(jnp.sqrt(sq), EPS)
    return jnp.tanh(norm) * u / norm


def _transform_body(x_ref, wt_ref, b_ref, z_ref):
    t = _logmap0(x_ref[...])
    y = lax.dot(t, wt_ref[...], preferred_element_type=jnp.float32) + b_ref[...]
    z_ref[...] = _expmap0(y)


def _transform_mid_body(p_ref, wt_ref, b_ref, z_ref):
    x = jnp.maximum(p_ref[0] + p_ref[1], 0.0)
    t = _logmap0(x)
    y = lax.dot(t, wt_ref[...], preferred_element_type=jnp.float32) + b_ref[...]
    z_ref[...] = _expmap0(y)


def _classifier_body(p_ref, wt_ref, b_ref, o_ref):
    x = jnp.maximum(p_ref[0] + p_ref[1], 0.0)
    t = _logmap0(x)
    o_ref[...] = lax.dot(t, wt_ref[...], preferred_element_type=jnp.float32) + b_ref[...]


def _tc_transform(x, wt, b):
    return pl.pallas_call(
        _transform_body,
        grid=(_N_BLKS,),
        in_specs=[
            pl.BlockSpec((_ROWS_PER_BLK, DIM), lambda i: (i, 0)),
            pl.BlockSpec((DIM, DIM), lambda i: (0, 0)),
            pl.BlockSpec((1, DIM), lambda i: (0, 0)),
        ],
        out_specs=pl.BlockSpec((_ROWS_PER_BLK, DIM), lambda i: (i, 0)),
        out_shape=jax.ShapeDtypeStruct((N_NODES, DIM), jnp.float32),
    )(x, wt, b.reshape(1, DIM))


def _tc_transform_mid(p, wt, b):
    return pl.pallas_call(
        _transform_mid_body,
        grid=(_N_BLKS,),
        in_specs=[
            pl.BlockSpec((2, _ROWS_PER_BLK, DIM), lambda i: (0, i, 0)),
            pl.BlockSpec((DIM, DIM), lambda i: (0, 0)),
            pl.BlockSpec((1, DIM), lambda i: (0, 0)),
        ],
        out_specs=pl.BlockSpec((_ROWS_PER_BLK, DIM), lambda i: (i, 0)),
        out_shape=jax.ShapeDtypeStruct((N_NODES, DIM), jnp.float32),
    )(p, wt, b.reshape(1, DIM))


def _tc_classifier(p, wt, b):
    return pl.pallas_call(
        _classifier_body,
        grid=(_N_BLKS,),
        in_specs=[
            pl.BlockSpec((2, _ROWS_PER_BLK, DIM), lambda i: (0, i, 0)),
            pl.BlockSpec((DIM, NUM_CLASSES), lambda i: (0, 0)),
            pl.BlockSpec((1, NUM_CLASSES), lambda i: (0, 0)),
        ],
        out_specs=pl.BlockSpec((_ROWS_PER_BLK, NUM_CLASSES), lambda i: (i, 0)),
        out_shape=jax.ShapeDtypeStruct((N_NODES, NUM_CLASSES), jnp.float32),
    )(p, wt, b.reshape(1, NUM_CLASSES))


def _sc_segment_sum(z, edge_index, zeros):
    mesh = plsc.VectorSubcoreMesh(core_axis_name="c", subcore_axis_name="s")

    @functools.partial(
        pl.kernel,
        mesh=mesh,
        out_type=jax.ShapeDtypeStruct((2, N_NODES, DIM), jnp.float32),
        scratch_types=[
            pltpu.VMEM((_CHUNK,), jnp.int32),
            pltpu.VMEM((_CHUNK,), jnp.int32),
            pltpu.VMEM((_CHUNK, DIM), jnp.float32),
            pltpu.VMEM_SHARED((N_NODES, DIM), jnp.float32),
            pltpu.SemaphoreType.DMA,
        ],
    )
    def seg(z_hbm, src_hbm, dst_hbm, zeros_hbm, out_hbm, idx_s, idx_d, rows,
            acc, sem):
        cid = lax.axis_index("c")
        sid = lax.axis_index("s")
        wid = sid * 2 + cid
        rbase = pl.multiple_of(sid * _STRIPE + jnp.minimum(sid, 2) * _STRIPE_EXTRA, 8)

        def stripe_copy(src_ref, dst_ref):
            pltpu.sync_copy(src_ref.at[pl.ds(rbase, _STRIPE)],
                            dst_ref.at[pl.ds(rbase, _STRIPE)])

            @pl.when(sid < 2)
            def _():
                tail = pl.multiple_of(rbase + _STRIPE, 8)
                pltpu.sync_copy(src_ref.at[pl.ds(tail, _STRIPE_EXTRA)],
                                dst_ref.at[pl.ds(tail, _STRIPE_EXTRA)])

        stripe_copy(zeros_hbm, acc)
        plsc.subcore_barrier()

        def do_chunk(c):
            off = pl.multiple_of(c * _CHUNK, _CHUNK)
            pltpu.sync_copy(src_hbm.at[pl.ds(off, _CHUNK)], idx_s)
            pltpu.sync_copy(dst_hbm.at[pl.ds(off, _CHUNK)], idx_d)
            pltpu.async_copy(z_hbm.at[idx_s], rows, sem).wait()
            pltpu.sync_copy(rows, acc.at[idx_d], add=True)

        def body(i, carry):
            do_chunk(wid * _FULL + i)
            return carry

        lax.fori_loop(0, _FULL, body, 0)

        @pl.when(wid < _EXTRA)
        def _():
            do_chunk(_NW * _FULL + wid)

        plsc.subcore_barrier()
        stripe_copy(acc, out_hbm.at[cid])

    return seg(z, edge_index[0], edge_index[1], zeros)


def kernel(edge_index, entity_embeddings, W1, b1, W2, b2, Wc, bc):
    zeros = jnp.zeros((N_NODES, DIM), jnp.float32)
    z1 = _tc_transform(entity_embeddings, W1.T, b1)
    p1 = _sc_segment_sum(z1, edge_index, zeros)
    z2 = _tc_transform_mid(p1, W2.T, b2)
    p2 = _sc_segment_sum(z2, edge_index, zeros)
    return _tc_classifier(p2, Wc.T, bc)


# speedup vs baseline: 10.3772x; 1.6861x over previous
import functools

import jax
import jax.numpy as jnp
from jax import lax
from jax.experimental import pallas as pl
from jax.experimental.pallas import tpu as pltpu
from jax.experimental.pallas import tpu_sc as plsc

N_NODES = 10000
N_EDGES = 320000
DIM = 128
NUM_CLASSES = 10
EPS = 1e-15

_ROWS_PER_BLK = 1000
_N_BLKS = N_NODES // _ROWS_PER_BLK

_CHUNK = 128
_K = 1
_SUPER = _K * _CHUNK
_N_CHUNKS = N_EDGES // _CHUNK
_NW = 32
_FULL = _N_CHUNKS // _NW
_EXTRA = _N_CHUNKS - _FULL * _NW
_NSUP = _FULL // _K
_NPAIR = _NSUP // 2
_STRIPE = 624
_STRIPE_EXTRA = 8


def _logmap0(x):
    sq = jnp.sum(x * x, axis=-1, keepdims=True)
    norm = jnp.maximum(jnp.sqrt(sq), EPS)
    arg = jnp.clip(norm, 0.0, 1.0 - 1e-6)
    att = 0.5 * jnp.log((1.0 + arg) / (1.0 - arg))
    return x * (att / norm)


def _expmap0(u):
    sq = jnp.sum(u * u, axis=-1, keepdims=True)
    norm = jnp.maximum(jnp.sqrt(sq), EPS)
    return jnp.tanh(norm) * u / norm


def _transform_body(x_ref, wt_ref, b_ref, z_ref):
    t = _logmap0(x_ref[...])
    y = lax.dot(t, wt_ref[...], preferred_element_type=jnp.float32) + b_ref[...]
    z_ref[...] = _expmap0(y)


def _transform_mid_body(p_ref, wt_ref, b_ref, z_ref):
    x = jnp.maximum(p_ref[0] + p_ref[1], 0.0)
    t = _logmap0(x)
    y = lax.dot(t, wt_ref[...], preferred_element_type=jnp.float32) + b_ref[...]
    z_ref[...] = _expmap0(y)


def _classifier_body(p_ref, wt_ref, b_ref, o_ref):
    x = jnp.maximum(p_ref[0] + p_ref[1], 0.0)
    t = _logmap0(x)
    o_ref[...] = lax.dot(t, wt_ref[...], preferred_element_type=jnp.float32) + b_ref[...]


def _tc_transform(x, wt, b):
    return pl.pallas_call(
        _transform_body,
        grid=(_N_BLKS,),
        in_specs=[
            pl.BlockSpec((_ROWS_PER_BLK, DIM), lambda i: (i, 0)),
            pl.BlockSpec((DIM, DIM), lambda i: (0, 0)),
            pl.BlockSpec((1, DIM), lambda i: (0, 0)),
        ],
        out_specs=pl.BlockSpec((_ROWS_PER_BLK, DIM), lambda i: (i, 0)),
        out_shape=jax.ShapeDtypeStruct((N_NODES, DIM), jnp.float32),
    )(x, wt, b.reshape(1, DIM))


def _tc_transform_mid(p, wt, b):
    return pl.pallas_call(
        _transform_mid_body,
        grid=(_N_BLKS,),
        in_specs=[
            pl.BlockSpec((2, _ROWS_PER_BLK, DIM), lambda i: (0, i, 0)),
            pl.BlockSpec((DIM, DIM), lambda i: (0, 0)),
            pl.BlockSpec((1, DIM), lambda i: (0, 0)),
        ],
        out_specs=pl.BlockSpec((_ROWS_PER_BLK, DIM), lambda i: (i, 0)),
        out_shape=jax.ShapeDtypeStruct((N_NODES, DIM), jnp.float32),
    )(p, wt, b.reshape(1, DIM))


def _tc_classifier(p, wt, b):
    return pl.pallas_call(
        _classifier_body,
        grid=(_N_BLKS,),
        in_specs=[
            pl.BlockSpec((2, _ROWS_PER_BLK, DIM), lambda i: (0, i, 0)),
            pl.BlockSpec((DIM, NUM_CLASSES), lambda i: (0, 0)),
            pl.BlockSpec((1, NUM_CLASSES), lambda i: (0, 0)),
        ],
        out_specs=pl.BlockSpec((_ROWS_PER_BLK, NUM_CLASSES), lambda i: (i, 0)),
        out_shape=jax.ShapeDtypeStruct((N_NODES, NUM_CLASSES), jnp.float32),
    )(p, wt, b.reshape(1, NUM_CLASSES))


def _sc_segment_sum(z, edge_index, zeros):
    mesh = plsc.VectorSubcoreMesh(core_axis_name="c", subcore_axis_name="s")

    @functools.partial(
        pl.kernel,
        mesh=mesh,
        out_type=jax.ShapeDtypeStruct((2, N_NODES, DIM), jnp.float32),
        scratch_types=[
            pltpu.VMEM((_SUPER,), jnp.int32),
            pltpu.VMEM((_SUPER,), jnp.int32),
            pltpu.VMEM((_CHUNK,), jnp.int32),
            pltpu.VMEM((_CHUNK,), jnp.int32),
            pltpu.VMEM((_SUPER, DIM), jnp.float32),
            pltpu.VMEM((_SUPER, DIM), jnp.float32),
            pltpu.VMEM_SHARED((N_NODES, DIM), jnp.float32),
            pltpu.SemaphoreType.DMA,
            pltpu.SemaphoreType.DMA,
            pltpu.SemaphoreType.DMA,
            pltpu.SemaphoreType.DMA,
            pltpu.SemaphoreType.DMA,
            pltpu.SemaphoreType.DMA,
        ],
    )
    def seg(z_hbm, src_hbm, dst_hbm, zeros_hbm, out_hbm,
            sb0, sb1, d00, d10, r0, r1, acc,
            si0, si1, sg0, sg1, ss0, ss1):
        sb = (sb0, sb1)
        db = ((d00,), (d10,))
        rows = (r0, r1)
        sem_i = (si0, si1)
        sem_g = (sg0, sg1)
        sem_s = (ss0, ss1)

        cid = lax.axis_index("c")
        sid = lax.axis_index("s")
        wid = sid * 2 + cid
        rbase = pl.multiple_of(sid * _STRIPE + jnp.minimum(sid, 2) * _STRIPE_EXTRA, 8)

        def stripe_copy(src_ref, dst_ref):
            pltpu.sync_copy(src_ref.at[pl.ds(rbase, _STRIPE)],
                            dst_ref.at[pl.ds(rbase, _STRIPE)])

            @pl.when(sid < 2)
            def _():
                tail = pl.multiple_of(rbase + _STRIPE, 8)
                pltpu.sync_copy(src_ref.at[pl.ds(tail, _STRIPE_EXTRA)],
                                dst_ref.at[pl.ds(tail, _STRIPE_EXTRA)])

        stripe_copy(zeros_hbm, acc)
        plsc.subcore_barrier()

        def idx_start(b, s):
            off = pl.multiple_of(s * _SUPER, 8)
            pltpu.async_copy(src_hbm.at[pl.ds(off, _SUPER)], sb[b], sem_i[b])
            for j in range(_K):
                pltpu.async_copy(dst_hbm.at[pl.ds(off + j * _CHUNK, _CHUNK)],
                                 db[b][j], sem_i[b])

        def idx_wait(b):
            pltpu.make_async_copy(src_hbm.at[pl.ds(0, _SUPER)], sb[b],
                                  sem_i[b]).wait()
            for j in range(_K):
                pltpu.make_async_copy(dst_hbm.at[pl.ds(0, _CHUNK)], db[b][j],
                                      sem_i[b]).wait()

        def gathers_start(b):
            for j in range(_K):
                pltpu.async_copy(z_hbm.at[sb[b].at[pl.ds(j * _CHUNK, _CHUNK)]],
                                 rows[b].at[pl.ds(j * _CHUNK, _CHUNK)],
                                 sem_g[b])

        def gathers_wait(b):
            for j in range(_K):
                pltpu.make_async_copy(
                    z_hbm.at[sb[b].at[pl.ds(j * _CHUNK, _CHUNK)]],
                    rows[b].at[pl.ds(j * _CHUNK, _CHUNK)], sem_g[b]).wait()

        def scatters_start(b):
            for j in range(_K):
                pltpu.async_copy(rows[b].at[pl.ds(j * _CHUNK, _CHUNK)],
                                 acc.at[db[b][j]], sem_s[b], add=True)

        def scatters_wait(b):
            for j in range(_K):
                pltpu.make_async_copy(rows[b].at[pl.ds(j * _CHUNK, _CHUNK)],
                                      acc.at[db[b][j]], sem_s[b]).wait()

        base_sup = wid * _NSUP
        idx_start(0, base_sup)

        def pair(t, carry):
            s0 = base_sup + 2 * t
            idx_wait(0)
            gathers_start(0)

            @pl.when(t > 0)
            def _():
                scatters_wait(1)

            idx_start(1, s0 + 1)
            gathers_wait(0)
            scatters_start(0)

            idx_wait(1)
            gathers_start(1)
            scatters_wait(0)

            @pl.when(t < _NPAIR - 1)
            def _():
                idx_start(0, s0 + 2)

            gathers_wait(1)
            scatters_start(1)
            return carry

        lax.fori_loop(0, _NPAIR, pair, 0)
        scatters_wait(1)

        @pl.when(wid < _EXTRA)
        def _():
            off = pl.multiple_of((_NW * _FULL + wid) * _CHUNK, 8)
            pltpu.sync_copy(src_hbm.at[pl.ds(off, _CHUNK)],
                            sb0.at[pl.ds(0, _CHUNK)])
            pltpu.sync_copy(dst_hbm.at[pl.ds(off, _CHUNK)], d00)
            pltpu.async_copy(z_hbm.at[sb0.at[pl.ds(0, _CHUNK)]],
                             r0.at[pl.ds(0, _CHUNK)], sg0).wait()
            pltpu.sync_copy(r0.at[pl.ds(0, _CHUNK)], acc.at[d00], add=True)

        plsc.subcore_barrier()
        stripe_copy(acc, out_hbm.at[cid])

    return seg(z, edge_index[0], edge_index[1], zeros)


def kernel(edge_index, entity_embeddings, W1, b1, W2, b2, Wc, bc):
    zeros = jnp.zeros((N_NODES, DIM), jnp.float32)
    z1 = _tc_transform(entity_embeddings, W1.T, b1)
    p1 = _sc_segment_sum(z1, edge_index, zeros)
    z2 = _tc_transform_mid(p1, W2.T, b2)
    p2 = _sc_segment_sum(z2, edge_index, zeros)
    return _tc_classifier(p2, Wc.T, bc)


# speedup vs baseline: 10.4597x; 1.0080x over previous
import functools

import jax
import jax.numpy as jnp
from jax import lax
from jax.experimental import pallas as pl
from jax.experimental.pallas import tpu as pltpu
from jax.experimental.pallas import tpu_sc as plsc

N_NODES = 10000
N_EDGES = 320000
DIM = 128
NUM_CLASSES = 10
EPS = 1e-15

_ROWS_PER_BLK = 1000
_N_BLKS = N_NODES // _ROWS_PER_BLK

_CHUNK = 128
_K = 1
_SUPER = _K * _CHUNK
_N_CHUNKS = N_EDGES // _CHUNK
_NW = 32
_FULL = _N_CHUNKS // _NW
_EXTRA = _N_CHUNKS - _FULL * _NW
_NSUP = _FULL // _K
_NTRIP = _NSUP // 3
_STRIPE = 624
_STRIPE_EXTRA = 8


def _logmap0(x):
    sq = jnp.sum(x * x, axis=-1, keepdims=True)
    norm = jnp.maximum(jnp.sqrt(sq), EPS)
    arg = jnp.clip(norm, 0.0, 1.0 - 1e-6)
    att = 0.5 * jnp.log((1.0 + arg) / (1.0 - arg))
    return x * (att / norm)


def _expmap0(u):
    sq = jnp.sum(u * u, axis=-1, keepdims=True)
    norm = jnp.maximum(jnp.sqrt(sq), EPS)
    return jnp.tanh(norm) * u / norm


def _transform_body(x_ref, wt_ref, b_ref, z_ref):
    t = _logmap0(x_ref[...])
    y = lax.dot(t, wt_ref[...], preferred_element_type=jnp.float32) + b_ref[...]
    z_ref[...] = _expmap0(y)


def _transform_mid_body(p_ref, wt_ref, b_ref, z_ref):
    x = jnp.maximum(p_ref[0] + p_ref[1], 0.0)
    t = _logmap0(x)
    y = lax.dot(t, wt_ref[...], preferred_element_type=jnp.float32) + b_ref[...]
    z_ref[...] = _expmap0(y)


def _classifier_body(p_ref, wt_ref, b_ref, o_ref):
    x = jnp.maximum(p_ref[0] + p_ref[1], 0.0)
    t = _logmap0(x)
    o_ref[...] = lax.dot(t, wt_ref[...], preferred_element_type=jnp.float32) + b_ref[...]


def _tc_transform(x, wt, b):
    return pl.pallas_call(
        _transform_body,
        grid=(_N_BLKS,),
        in_specs=[
            pl.BlockSpec((_ROWS_PER_BLK, DIM), lambda i: (i, 0)),
            pl.BlockSpec((DIM, DIM), lambda i: (0, 0)),
            pl.BlockSpec((1, DIM), lambda i: (0, 0)),
        ],
        out_specs=pl.BlockSpec((_ROWS_PER_BLK, DIM), lambda i: (i, 0)),
        out_shape=jax.ShapeDtypeStruct((N_NODES, DIM), jnp.float32),
    )(x, wt, b.reshape(1, DIM))


def _tc_transform_mid(p, wt, b):
    return pl.pallas_call(
        _transform_mid_body,
        grid=(_N_BLKS,),
        in_specs=[
            pl.BlockSpec((2, _ROWS_PER_BLK, DIM), lambda i: (0, i, 0)),
            pl.BlockSpec((DIM, DIM), lambda i: (0, 0)),
            pl.BlockSpec((1, DIM), lambda i: (0, 0)),
        ],
        out_specs=pl.BlockSpec((_ROWS_PER_BLK, DIM), lambda i: (i, 0)),
        out_shape=jax.ShapeDtypeStruct((N_NODES, DIM), jnp.float32),
    )(p, wt, b.reshape(1, DIM))


def _tc_classifier(p, wt, b):
    return pl.pallas_call(
        _classifier_body,
        grid=(_N_BLKS,),
        in_specs=[
            pl.BlockSpec((2, _ROWS_PER_BLK, DIM), lambda i: (0, i, 0)),
            pl.BlockSpec((DIM, NUM_CLASSES), lambda i: (0, 0)),
            pl.BlockSpec((1, NUM_CLASSES), lambda i: (0, 0)),
        ],
        out_specs=pl.BlockSpec((_ROWS_PER_BLK, NUM_CLASSES), lambda i: (i, 0)),
        out_shape=jax.ShapeDtypeStruct((N_NODES, NUM_CLASSES), jnp.float32),
    )(p, wt, b.reshape(1, NUM_CLASSES))


def _sc_segment_sum(z, edge_index, zeros):
    mesh = plsc.VectorSubcoreMesh(core_axis_name="c", subcore_axis_name="s")

    @functools.partial(
        pl.kernel,
        mesh=mesh,
        out_type=jax.ShapeDtypeStruct((2, N_NODES, DIM), jnp.float32),
        scratch_types=[
            pltpu.VMEM((_SUPER,), jnp.int32),
            pltpu.VMEM((_SUPER,), jnp.int32),
            pltpu.VMEM((_SUPER,), jnp.int32),
            pltpu.VMEM((_CHUNK,), jnp.int32),
            pltpu.VMEM((_CHUNK,), jnp.int32),
            pltpu.VMEM((_CHUNK,), jnp.int32),
            pltpu.VMEM((_SUPER, DIM), jnp.float32),
            pltpu.VMEM((_SUPER, DIM), jnp.float32),
            pltpu.VMEM((_SUPER, DIM), jnp.float32),
            pltpu.VMEM_SHARED((N_NODES, DIM), jnp.float32),
            pltpu.SemaphoreType.DMA,
            pltpu.SemaphoreType.DMA,
            pltpu.SemaphoreType.DMA,
            pltpu.SemaphoreType.DMA,
            pltpu.SemaphoreType.DMA,
            pltpu.SemaphoreType.DMA,
            pltpu.SemaphoreType.DMA,
            pltpu.SemaphoreType.DMA,
            pltpu.SemaphoreType.DMA,
        ],
    )
    def seg(z_hbm, src_hbm, dst_hbm, zeros_hbm, out_hbm,
            sb0, sb1, sb2, d00, d10, d20, r0, r1, r2, acc,
            si0, si1, si2, sg0, sg1, sg2, ss0, ss1, ss2):
        sb = (sb0, sb1, sb2)
        db = ((d00,), (d10,), (d20,))
        rows = (r0, r1, r2)
        sem_i = (si0, si1, si2)
        sem_g = (sg0, sg1, sg2)
        sem_s = (ss0, ss1, ss2)

        cid = lax.axis_index("c")
        sid = lax.axis_index("s")
        wid = sid * 2 + cid
        rbase = pl.multiple_of(sid * _STRIPE + jnp.minimum(sid, 2) * _STRIPE_EXTRA, 8)

        def stripe_copy(src_ref, dst_ref):
            pltpu.sync_copy(src_ref.at[pl.ds(rbase, _STRIPE)],
                            dst_ref.at[pl.ds(rbase, _STRIPE)])

            @pl.when(sid < 2)
            def _():
                tail = pl.multiple_of(rbase + _STRIPE, 8)
                pltpu.sync_copy(src_ref.at[pl.ds(tail, _STRIPE_EXTRA)],
                                dst_ref.at[pl.ds(tail, _STRIPE_EXTRA)])

        stripe_copy(zeros_hbm, acc)
        plsc.subcore_barrier()

        def idx_start(b, s):
            off = pl.multiple_of(s * _SUPER, 8)
            pltpu.async_copy(src_hbm.at[pl.ds(off, _SUPER)], sb[b], sem_i[b])
            for j in range(_K):
                pltpu.async_copy(dst_hbm.at[pl.ds(off + j * _CHUNK, _CHUNK)],
                                 db[b][j], sem_i[b])

        def idx_wait(b):
            pltpu.make_async_copy(src_hbm.at[pl.ds(0, _SUPER)], sb[b],
                                  sem_i[b]).wait()
            for j in range(_K):
                pltpu.make_async_copy(dst_hbm.at[pl.ds(0, _CHUNK)], db[b][j],
                                      sem_i[b]).wait()

        def gathers_start(b):
            for j in range(_K):
                pltpu.async_copy(z_hbm.at[sb[b].at[pl.ds(j * _CHUNK, _CHUNK)]],
                                 rows[b].at[pl.ds(j * _CHUNK, _CHUNK)],
                                 sem_g[b])

        def gathers_wait(b):
            for j in range(_K):
                pltpu.make_async_copy(
                    z_hbm.at[sb[b].at[pl.ds(j * _CHUNK, _CHUNK)]],
                    rows[b].at[pl.ds(j * _CHUNK, _CHUNK)], sem_g[b]).wait()

        def scatters_start(b):
            for j in range(_K):
                pltpu.async_copy(rows[b].at[pl.ds(j * _CHUNK, _CHUNK)],
                                 acc.at[db[b][j]], sem_s[b], add=True)

        def scatters_wait(b):
            for j in range(_K):
                pltpu.make_async_copy(rows[b].at[pl.ds(j * _CHUNK, _CHUNK)],
                                      acc.at[db[b][j]], sem_s[b]).wait()

        base_sup = wid * _NSUP
        idx_start(0, base_sup)

        def triple(t, carry):
            s0 = base_sup + 3 * t
            for b in range(3):
                bn = (b + 1) % 3
                idx_wait(b)
                gathers_start(b)
                if b == 2:
                    scatters_wait(0)
                else:
                    @pl.when(t > 0)
                    def _(bn=bn):
                        scatters_wait(bn)
                if b == 2:
                    @pl.when(t < _NTRIP - 1)
                    def _():
                        idx_start(0, s0 + 3)
                else:
                    idx_start(bn, s0 + b + 1)
                gathers_wait(b)
                scatters_start(b)
            return carry

        lax.fori_loop(0, _NTRIP, triple, 0)
        scatters_wait(1)
        scatters_wait(2)

        @pl.when(wid < _EXTRA)
        def _():
            off = pl.multiple_of((_NW * _FULL + wid) * _CHUNK, 8)
            pltpu.sync_copy(src_hbm.at[pl.ds(off, _CHUNK)],
                            sb0.at[pl.ds(0, _CHUNK)])
            pltpu.sync_copy(dst_hbm.at[pl.ds(off, _CHUNK)], d00)
            pltpu.async_copy(z_hbm.at[sb0.at[pl.ds(0, _CHUNK)]],
                             r0.at[pl.ds(0, _CHUNK)], sg0).wait()
            pltpu.sync_copy(r0.at[pl.ds(0, _CHUNK)], acc.at[d00], add=True)

        plsc.subcore_barrier()
        stripe_copy(acc, out_hbm.at[cid])

    return seg(z, edge_index[0], edge_index[1], zeros)


def kernel(edge_index, entity_embeddings, W1, b1, W2, b2, Wc, bc):
    zeros = jnp.zeros((N_NODES, DIM), jnp.float32)
    z1 = _tc_transform(entity_embeddings, W1.T, b1)
    p1 = _sc_segment_sum(z1, edge_index, zeros)
    z2 = _tc_transform_mid(p1, W2.T, b2)
    p2 = _sc_segment_sum(z2, edge_index, zeros)
    return _tc_classifier(p2, Wc.T, bc)


# speedup vs baseline: 10.7655x; 1.0292x over previous
import functools

import jax
import jax.numpy as jnp
from jax import lax
from jax.experimental import pallas as pl
from jax.experimental.pallas import tpu as pltpu
from jax.experimental.pallas import tpu_sc as plsc

N_NODES = 10000
N_EDGES = 320000
DIM = 128
NUM_CLASSES = 10
EPS = 1e-15

_ROWS_PER_BLK = 1000
_N_BLKS = N_NODES // _ROWS_PER_BLK

_CHUNK = 128
_K = 1
_SUPER = _K * _CHUNK
_N_CHUNKS = N_EDGES // _CHUNK
_NW = 32
_FULL = _N_CHUNKS // _NW
_EXTRA = _N_CHUNKS - _FULL * _NW
_NSUP = _FULL // _K
_NTRIP = _NSUP // 3
_STRIPE = 624
_STRIPE_EXTRA = 8


def _logmap0(x):
    sq = jnp.sum(x * x, axis=-1, keepdims=True)
    norm = jnp.maximum(jnp.sqrt(sq), EPS)
    arg = jnp.clip(norm, 0.0, 1.0 - 1e-6)
    att = 0.5 * jnp.log((1.0 + arg) / (1.0 - arg))
    return x * (att / norm)


def _expmap0(u):
    sq = jnp.sum(u * u, axis=-1, keepdims=True)
    norm = jnp.maximum(jnp.sqrt(sq), EPS)
    return jnp.tanh(norm) * u / norm


def _transform_body(x_ref, wt_ref, b_ref, z_ref):
    t = _logmap0(x_ref[...])
    y = lax.dot(t, wt_ref[...], preferred_element_type=jnp.float32) + b_ref[...]
    z_ref[...] = _expmap0(y)


def _transform_mid_body(p_ref, wt_ref, b_ref, z_ref):
    x = jnp.maximum(p_ref[0] + p_ref[1], 0.0)
    t = _logmap0(x)
    y = lax.dot(t, wt_ref[...], preferred_element_type=jnp.float32) + b_ref[...]
    z_ref[...] = _expmap0(y)


def _classifier_body(p_ref, wt_ref, b_ref, o_ref):
    x = jnp.maximum(p_ref[0] + p_ref[1], 0.0)
    t = _logmap0(x)
    o_ref[...] = lax.dot(t, wt_ref[...], preferred_element_type=jnp.float32) + b_ref[...]


def _tc_transform(x, wt, b):
    return pl.pallas_call(
        _transform_body,
        grid=(_N_BLKS,),
        in_specs=[
            pl.BlockSpec((_ROWS_PER_BLK, DIM), lambda i: (i, 0)),
            pl.BlockSpec((DIM, DIM), lambda i: (0, 0)),
            pl.BlockSpec((1, DIM), lambda i: (0, 0)),
        ],
        out_specs=pl.BlockSpec((_ROWS_PER_BLK, DIM), lambda i: (i, 0)),
        out_shape=jax.ShapeDtypeStruct((N_NODES, DIM), jnp.float32),
    )(x, wt, b.reshape(1, DIM))


def _tc_transform_mid(p, wt, b):
    return pl.pallas_call(
        _transform_mid_body,
        grid=(_N_BLKS,),
        in_specs=[
            pl.BlockSpec((2, _ROWS_PER_BLK, DIM), lambda i: (0, i, 0)),
            pl.BlockSpec((DIM, DIM), lambda i: (0, 0)),
            pl.BlockSpec((1, DIM), lambda i: (0, 0)),
        ],
        out_specs=pl.BlockSpec((_ROWS_PER_BLK, DIM), lambda i: (i, 0)),
        out_shape=jax.ShapeDtypeStruct((N_NODES, DIM), jnp.float32),
    )(p, wt, b.reshape(1, DIM))


def _tc_classifier(p, wt, b):
    return pl.pallas_call(
        _classifier_body,
        grid=(_N_BLKS,),
        in_specs=[
            pl.BlockSpec((2, _ROWS_PER_BLK, DIM), lambda i: (0, i, 0)),
            pl.BlockSpec((DIM, NUM_CLASSES), lambda i: (0, 0)),
            pl.BlockSpec((1, NUM_CLASSES), lambda i: (0, 0)),
        ],
        out_specs=pl.BlockSpec((_ROWS_PER_BLK, NUM_CLASSES), lambda i: (i, 0)),
        out_shape=jax.ShapeDtypeStruct((N_NODES, NUM_CLASSES), jnp.float32),
    )(p, wt, b.reshape(1, NUM_CLASSES))


def _sc_segment_sum(z, edge_index, zeros):
    mesh = plsc.VectorSubcoreMesh(core_axis_name="c", subcore_axis_name="s")

    @functools.partial(
        pl.kernel,
        mesh=mesh,
        out_type=jax.ShapeDtypeStruct((2, N_NODES, DIM), jnp.float32),
        scratch_types=[
            pltpu.VMEM((_SUPER,), jnp.int32),
            pltpu.VMEM((_SUPER,), jnp.int32),
            pltpu.VMEM((_SUPER,), jnp.int32),
            pltpu.VMEM((_CHUNK,), jnp.int32),
            pltpu.VMEM((_CHUNK,), jnp.int32),
            pltpu.VMEM((_CHUNK,), jnp.int32),
            pltpu.VMEM((_SUPER, DIM), jnp.float32),
            pltpu.VMEM((_SUPER, DIM), jnp.float32),
            pltpu.VMEM((_SUPER, DIM), jnp.float32),
            pltpu.VMEM_SHARED((N_NODES, DIM), jnp.float32),
            pltpu.SemaphoreType.DMA,
            pltpu.SemaphoreType.DMA,
            pltpu.SemaphoreType.DMA,
            pltpu.SemaphoreType.DMA,
            pltpu.SemaphoreType.DMA,
            pltpu.SemaphoreType.DMA,
            pltpu.SemaphoreType.DMA,
            pltpu.SemaphoreType.DMA,
            pltpu.SemaphoreType.DMA,
        ],
    )
    def seg(z_hbm, ei_hbm, zeros_hbm, out_hbm,
            sb0, sb1, sb2, d00, d10, d20, r0, r1, r2, acc,
            si0, si1, si2, sg0, sg1, sg2, ss0, ss1, ss2):
        sb = (sb0, sb1, sb2)
        db = ((d00,), (d10,), (d20,))
        rows = (r0, r1, r2)
        sem_i = (si0, si1, si2)
        sem_g = (sg0, sg1, sg2)
        sem_s = (ss0, ss1, ss2)

        cid = lax.axis_index("c")
        sid = lax.axis_index("s")
        wid = sid * 2 + cid
        rbase = pl.multiple_of(sid * _STRIPE + jnp.minimum(sid, 2) * _STRIPE_EXTRA, 8)

        def stripe_copy(src_ref, dst_ref):
            pltpu.sync_copy(src_ref.at[pl.ds(rbase, _STRIPE)],
                            dst_ref.at[pl.ds(rbase, _STRIPE)])

            @pl.when(sid < 2)
            def _():
                tail = pl.multiple_of(rbase + _STRIPE, 8)
                pltpu.sync_copy(src_ref.at[pl.ds(tail, _STRIPE_EXTRA)],
                                dst_ref.at[pl.ds(tail, _STRIPE_EXTRA)])

        stripe_copy(zeros_hbm, acc)
        plsc.subcore_barrier()

        def idx_start(b, s):
            off = pl.multiple_of(s * _SUPER, 8)
            pltpu.async_copy(ei_hbm.at[pl.ds(off, _SUPER)], sb[b], sem_i[b])
            for j in range(_K):
                pltpu.async_copy(
                    ei_hbm.at[pl.ds(N_EDGES + off + j * _CHUNK, _CHUNK)],
                    db[b][j], sem_i[b])

        def idx_wait(b):
            pltpu.make_async_copy(ei_hbm.at[pl.ds(0, _SUPER)], sb[b],
                                  sem_i[b]).wait()
            for j in range(_K):
                pltpu.make_async_copy(ei_hbm.at[pl.ds(0, _CHUNK)], db[b][j],
                                      sem_i[b]).wait()

        def gathers_start(b):
            for j in range(_K):
                pltpu.async_copy(z_hbm.at[sb[b].at[pl.ds(j * _CHUNK, _CHUNK)]],
                                 rows[b].at[pl.ds(j * _CHUNK, _CHUNK)],
                                 sem_g[b])

        def gathers_wait(b):
            for j in range(_K):
                pltpu.make_async_copy(
                    z_hbm.at[sb[b].at[pl.ds(j * _CHUNK, _CHUNK)]],
                    rows[b].at[pl.ds(j * _CHUNK, _CHUNK)], sem_g[b]).wait()

        def scatters_start(b):
            for j in range(_K):
                pltpu.async_copy(rows[b].at[pl.ds(j * _CHUNK, _CHUNK)],
                                 acc.at[db[b][j]], sem_s[b], add=True)

        def scatters_wait(b):
            for j in range(_K):
                pltpu.make_async_copy(rows[b].at[pl.ds(j * _CHUNK, _CHUNK)],
                                      acc.at[db[b][j]], sem_s[b]).wait()

        base_sup = wid * _NSUP
        idx_start(0, base_sup)

        def triple(t, carry):
            s0 = base_sup + 3 * t
            for b in range(3):
                bn = (b + 1) % 3
                idx_wait(b)
                gathers_start(b)
                if b == 2:
                    scatters_wait(0)
                else:
                    @pl.when(t > 0)
                    def _(bn=bn):
                        scatters_wait(bn)
                if b == 2:
                    @pl.when(t < _NTRIP - 1)
                    def _():
                        idx_start(0, s0 + 3)
                else:
                    idx_start(bn, s0 + b + 1)
                gathers_wait(b)
                scatters_start(b)
            return carry

        lax.fori_loop(0, _NTRIP, triple, 0)
        scatters_wait(1)
        scatters_wait(2)

        @pl.when(wid < _EXTRA)
        def _():
            off = pl.multiple_of((_NW * _FULL + wid) * _CHUNK, 8)
            pltpu.sync_copy(ei_hbm.at[pl.ds(off, _CHUNK)],
                            sb0.at[pl.ds(0, _CHUNK)])
            pltpu.sync_copy(ei_hbm.at[pl.ds(N_EDGES + off, _CHUNK)], d00)
            pltpu.async_copy(z_hbm.at[sb0.at[pl.ds(0, _CHUNK)]],
                             r0.at[pl.ds(0, _CHUNK)], sg0).wait()
            pltpu.sync_copy(r0.at[pl.ds(0, _CHUNK)], acc.at[d00], add=True)

        plsc.subcore_barrier()
        stripe_copy(acc, out_hbm.at[cid])

    return seg(z, edge_index.reshape(2 * N_EDGES), zeros)


def kernel(edge_index, entity_embeddings, W1, b1, W2, b2, Wc, bc):
    zeros = jnp.zeros((N_NODES, DIM), jnp.float32)
    z1 = _tc_transform(entity_embeddings, W1.T, b1)
    p1 = _sc_segment_sum(z1, edge_index, zeros)
    z2 = _tc_transform_mid(p1, W2.T, b2)
    p2 = _sc_segment_sum(z2, edge_index, zeros)
    return _tc_classifier(p2, Wc.T, bc)


# speedup vs baseline: 10.8708x; 1.0098x over previous
import functools

import jax
import jax.numpy as jnp
from jax import lax
from jax.experimental import pallas as pl
from jax.experimental.pallas import tpu as pltpu
from jax.experimental.pallas import tpu_sc as plsc

N_NODES = 10000
N_EDGES = 320000
DIM = 128
NUM_CLASSES = 10
EPS = 1e-15

_ROWS_PER_BLK = 1000
_N_BLKS = N_NODES // _ROWS_PER_BLK

_CHUNK = 128
_K = 1
_SUPER = _K * _CHUNK
_N_CHUNKS = N_EDGES // _CHUNK
_NW = 32
_FULL = _N_CHUNKS // _NW
_EXTRA = _N_CHUNKS - _FULL * _NW
_NSUP = _FULL // _K
_NTRIP = _NSUP // 3
_STRIPE = 624
_STRIPE_EXTRA = 8


def _logmap0(x):
    sq = jnp.sum(x * x, axis=-1, keepdims=True)
    norm = jnp.maximum(jnp.sqrt(sq), EPS)
    arg = jnp.clip(norm, 0.0, 1.0 - 1e-6)
    att = 0.5 * jnp.log((1.0 + arg) / (1.0 - arg))
    return x * (att / norm)


def _expmap0(u):
    sq = jnp.sum(u * u, axis=-1, keepdims=True)
    norm = jnp.maximum(jnp.sqrt(sq), EPS)
    return jnp.tanh(norm) * u / norm


def _logmap0_scale_t(xt):
    sq = jnp.sum(xt * xt, axis=0, keepdims=True)
    inv = lax.rsqrt(jnp.maximum(sq, EPS * EPS))
    norm = sq * inv
    arg = jnp.clip(norm, 0.0, 1.0 - 1e-6)
    att = 0.5 * jnp.log((1.0 + arg) / (1.0 - arg))
    return att * inv


def _expmap0_scale_t(ut):
    sq = jnp.sum(ut * ut, axis=0, keepdims=True)
    inv = lax.rsqrt(jnp.maximum(sq, EPS * EPS))
    return jnp.tanh(sq * inv) * inv


def _transform_t(xt, w_ref, bc_ref):
    t = xt * _logmap0_scale_t(xt)
    yt = lax.dot(w_ref[...], t, preferred_element_type=jnp.float32) + bc_ref[...]
    return yt * _expmap0_scale_t(yt)


def _transform_body(x_ref, w_ref, bc_ref, z_ref):
    xt = x_ref[...].T
    z_ref[...] = _transform_t(xt, w_ref, bc_ref).T


def _transform_mid_body(p_ref, w_ref, bc_ref, z_ref):
    x = jnp.maximum(p_ref[0] + p_ref[1], 0.0)
    z_ref[...] = _transform_t(x.T, w_ref, bc_ref).T


def _classifier_body(p_ref, wt_ref, b_ref, o_ref):
    x = jnp.maximum(p_ref[0] + p_ref[1], 0.0)
    xt = x.T
    t = (xt * _logmap0_scale_t(xt)).T
    o_ref[...] = lax.dot(t, wt_ref[...], preferred_element_type=jnp.float32) + b_ref[...]


def _tc_transform(x, w, b):
    return pl.pallas_call(
        _transform_body,
        grid=(_N_BLKS,),
        in_specs=[
            pl.BlockSpec((_ROWS_PER_BLK, DIM), lambda i: (i, 0)),
            pl.BlockSpec((DIM, DIM), lambda i: (0, 0)),
            pl.BlockSpec((DIM, 1), lambda i: (0, 0)),
        ],
        out_specs=pl.BlockSpec((_ROWS_PER_BLK, DIM), lambda i: (i, 0)),
        out_shape=jax.ShapeDtypeStruct((N_NODES, DIM), jnp.float32),
    )(x, w, b.reshape(DIM, 1))


def _tc_transform_mid(p, w, b):
    return pl.pallas_call(
        _transform_mid_body,
        grid=(_N_BLKS,),
        in_specs=[
            pl.BlockSpec((2, _ROWS_PER_BLK, DIM), lambda i: (0, i, 0)),
            pl.BlockSpec((DIM, DIM), lambda i: (0, 0)),
            pl.BlockSpec((DIM, 1), lambda i: (0, 0)),
        ],
        out_specs=pl.BlockSpec((_ROWS_PER_BLK, DIM), lambda i: (i, 0)),
        out_shape=jax.ShapeDtypeStruct((N_NODES, DIM), jnp.float32),
    )(p, w, b.reshape(DIM, 1))


def _tc_classifier(p, wt, b):
    return pl.pallas_call(
        _classifier_body,
        grid=(_N_BLKS,),
        in_specs=[
            pl.BlockSpec((2, _ROWS_PER_BLK, DIM), lambda i: (0, i, 0)),
            pl.BlockSpec((DIM, NUM_CLASSES), lambda i: (0, 0)),
            pl.BlockSpec((1, NUM_CLASSES), lambda i: (0, 0)),
        ],
        out_specs=pl.BlockSpec((_ROWS_PER_BLK, NUM_CLASSES), lambda i: (i, 0)),
        out_shape=jax.ShapeDtypeStruct((N_NODES, NUM_CLASSES), jnp.float32),
    )(p, wt, b.reshape(1, NUM_CLASSES))


def _sc_segment_sum(z, edge_index, zeros):
    mesh = plsc.VectorSubcoreMesh(core_axis_name="c", subcore_axis_name="s")

    @functools.partial(
        pl.kernel,
        mesh=mesh,
        out_type=jax.ShapeDtypeStruct((2, N_NODES, DIM), jnp.float32),
        scratch_types=[
            pltpu.VMEM((_SUPER,), jnp.int32),
            pltpu.VMEM((_SUPER,), jnp.int32),
            pltpu.VMEM((_SUPER,), jnp.int32),
            pltpu.VMEM((_CHUNK,), jnp.int32),
            pltpu.VMEM((_CHUNK,), jnp.int32),
            pltpu.VMEM((_CHUNK,), jnp.int32),
            pltpu.VMEM((_SUPER, DIM), jnp.float32),
            pltpu.VMEM((_SUPER, DIM), jnp.float32),
            pltpu.VMEM((_SUPER, DIM), jnp.float32),
            pltpu.VMEM_SHARED((N_NODES, DIM), jnp.float32),
            pltpu.SemaphoreType.DMA,
            pltpu.SemaphoreType.DMA,
            pltpu.SemaphoreType.DMA,
            pltpu.SemaphoreType.DMA,
            pltpu.SemaphoreType.DMA,
            pltpu.SemaphoreType.DMA,
            pltpu.SemaphoreType.DMA,
            pltpu.SemaphoreType.DMA,
            pltpu.SemaphoreType.DMA,
        ],
    )
    def seg(z_hbm, ei_hbm, zeros_hbm, out_hbm,
            sb0, sb1, sb2, d00, d10, d20, r0, r1, r2, acc,
            si0, si1, si2, sg0, sg1, sg2, ss0, ss1, ss2):
        sb = (sb0, sb1, sb2)
        db = ((d00,), (d10,), (d20,))
        rows = (r0, r1, r2)
        sem_i = (si0, si1, si2)
        sem_g = (sg0, sg1, sg2)
        sem_s = (ss0, ss1, ss2)

        cid = lax.axis_index("c")
        sid = lax.axis_index("s")
        wid = sid * 2 + cid
        rbase = pl.multiple_of(sid * _STRIPE + jnp.minimum(sid, 2) * _STRIPE_EXTRA, 8)

        def stripe_copy(src_ref, dst_ref):
            pltpu.sync_copy(src_ref.at[pl.ds(rbase, _STRIPE)],
                            dst_ref.at[pl.ds(rbase, _STRIPE)])

            @pl.when(sid < 2)
            def _():
                tail = pl.multiple_of(rbase + _STRIPE, 8)
                pltpu.sync_copy(src_ref.at[pl.ds(tail, _STRIPE_EXTRA)],
                                dst_ref.at[pl.ds(tail, _STRIPE_EXTRA)])

        stripe_copy(zeros_hbm, acc)
        plsc.subcore_barrier()

        def idx_start(b, s):
            off = pl.multiple_of(s * _SUPER, 8)
            pltpu.async_copy(ei_hbm.at[pl.ds(off, _SUPER)], sb[b], sem_i[b])
            for j in range(_K):
                pltpu.async_copy(
                    ei_hbm.at[pl.ds(N_EDGES + off + j * _CHUNK, _CHUNK)],
                    db[b][j], sem_i[b])

        def idx_wait(b):
            pltpu.make_async_copy(ei_hbm.at[pl.ds(0, _SUPER)], sb[b],
                                  sem_i[b]).wait()
            for j in range(_K):
                pltpu.make_async_copy(ei_hbm.at[pl.ds(0, _CHUNK)], db[b][j],
                                      sem_i[b]).wait()

        def gathers_start(b):
            for j in range(_K):
                pltpu.async_copy(z_hbm.at[sb[b].at[pl.ds(j * _CHUNK, _CHUNK)]],
                                 rows[b].at[pl.ds(j * _CHUNK, _CHUNK)],
                                 sem_g[b])

        def gathers_wait(b):
            for j in range(_K):
                pltpu.make_async_copy(
                    z_hbm.at[sb[b].at[pl.ds(j * _CHUNK, _CHUNK)]],
                    rows[b].at[pl.ds(j * _CHUNK, _CHUNK)], sem_g[b]).wait()

        def scatters_start(b):
            for j in range(_K):
                pltpu.async_copy(rows[b].at[pl.ds(j * _CHUNK, _CHUNK)],
                                 acc.at[db[b][j]], sem_s[b], add=True)

        def scatters_wait(b):
            for j in range(_K):
                pltpu.make_async_copy(rows[b].at[pl.ds(j * _CHUNK, _CHUNK)],
                                      acc.at[db[b][j]], sem_s[b]).wait()

        base_sup = wid * _NSUP
        idx_start(0, base_sup)

        def triple(t, carry):
            s0 = base_sup + 3 * t
            for b in range(3):
                bn = (b + 1) % 3
                idx_wait(b)
                gathers_start(b)
                if b == 2:
                    scatters_wait(0)
                else:
                    @pl.when(t > 0)
                    def _(bn=bn):
                        scatters_wait(bn)
                if b == 2:
                    @pl.when(t < _NTRIP - 1)
                    def _():
                        idx_start(0, s0 + 3)
                else:
                    idx_start(bn, s0 + b + 1)
                gathers_wait(b)
                scatters_start(b)
            return carry

        lax.fori_loop(0, _NTRIP, triple, 0)
        scatters_wait(1)
        scatters_wait(2)

        @pl.when(wid < _EXTRA)
        def _():
            off = pl.multiple_of((_NW * _FULL + wid) * _CHUNK, 8)
            pltpu.sync_copy(ei_hbm.at[pl.ds(off, _CHUNK)],
                            sb0.at[pl.ds(0, _CHUNK)])
            pltpu.sync_copy(ei_hbm.at[pl.ds(N_EDGES + off, _CHUNK)], d00)
            pltpu.async_copy(z_hbm.at[sb0.at[pl.ds(0, _CHUNK)]],
                             r0.at[pl.ds(0, _CHUNK)], sg0).wait()
            pltpu.sync_copy(r0.at[pl.ds(0, _CHUNK)], acc.at[d00], add=True)

        plsc.subcore_barrier()
        stripe_copy(acc, out_hbm.at[cid])

    return seg(z, edge_index.reshape(2 * N_EDGES), zeros)


def kernel(edge_index, entity_embeddings, W1, b1, W2, b2, Wc, bc):
    zeros = jnp.zeros((N_NODES, DIM), jnp.float32)
    z1 = _tc_transform(entity_embeddings, W1, b1)
    p1 = _sc_segment_sum(z1, edge_index, zeros)
    z2 = _tc_transform_mid(p1, W2, b2)
    p2 = _sc_segment_sum(z2, edge_index, zeros)
    return _tc_classifier(p2, Wc.T, bc)


# speedup vs baseline: 12.8425x; 1.1814x over previous
import functools

import jax
import jax.numpy as jnp
from jax import lax
from jax.experimental import pallas as pl
from jax.experimental.pallas import tpu as pltpu
from jax.experimental.pallas import tpu_sc as plsc

N_NODES = 10000
N_EDGES = 320000
DIM = 128
NUM_CLASSES = 10
EPS = 1e-15

_ROWS_PER_BLK = 1000
_N_BLKS = N_NODES // _ROWS_PER_BLK

_CHUNK = 128
_K = 1
_SUPER = _K * _CHUNK
_N_CHUNKS = N_EDGES // _CHUNK
_NW = 32
_FULL = _N_CHUNKS // _NW
_EXTRA = _N_CHUNKS - _FULL * _NW
_NSUP = _FULL // _K
_NTRIP = _NSUP // 3
_STRIPE = 624
_STRIPE_EXTRA = 8


def _logmap0(x):
    sq = jnp.sum(x * x, axis=-1, keepdims=True)
    norm = jnp.maximum(jnp.sqrt(sq), EPS)
    arg = jnp.clip(norm, 0.0, 1.0 - 1e-6)
    att = 0.5 * jnp.log((1.0 + arg) / (1.0 - arg))
    return x * (att / norm)


def _expmap0(u):
    sq = jnp.sum(u * u, axis=-1, keepdims=True)
    norm = jnp.maximum(jnp.sqrt(sq), EPS)
    return jnp.tanh(norm) * u / norm


def _logmap0_scale_t(xt):
    sq = jnp.sum(xt * xt, axis=0, keepdims=True)
    inv = lax.rsqrt(jnp.maximum(sq, EPS * EPS))
    norm = sq * inv
    arg = jnp.clip(norm, 0.0, 1.0 - 1e-6)
    att = 0.5 * jnp.log((1.0 + arg) / (1.0 - arg))
    return att * inv


def _expmap0_scale_t(ut):
    sq = jnp.sum(ut * ut, axis=0, keepdims=True)
    inv = lax.rsqrt(jnp.maximum(sq, EPS * EPS))
    return jnp.tanh(sq * inv) * inv


def _transform_t(xt, w_ref, bc_ref):
    t = xt * _logmap0_scale_t(xt)
    yt = lax.dot(w_ref[...], t, preferred_element_type=jnp.float32) + bc_ref[...]
    return yt * _expmap0_scale_t(yt)


def _transform_body(x_ref, w_ref, bc_ref, z_ref):
    xt = x_ref[...].T
    z_ref[...] = _transform_t(xt, w_ref, bc_ref).T


def _transform_mid_body(p_ref, w_ref, bc_ref, z_ref):
    x = jnp.maximum(p_ref[0] + p_ref[1], 0.0)
    z_ref[...] = _transform_t(x.T, w_ref, bc_ref).T


def _classifier_body(p_ref, wt_ref, b_ref, o_ref):
    x = jnp.maximum(p_ref[0] + p_ref[1], 0.0)
    xt = x.T
    t = (xt * _logmap0_scale_t(xt)).T
    o_ref[...] = lax.dot(t, wt_ref[...], preferred_element_type=jnp.float32) + b_ref[...]


def _tc_transform(x, w, b):
    return pl.pallas_call(
        _transform_body,
        grid=(_N_BLKS,),
        in_specs=[
            pl.BlockSpec((_ROWS_PER_BLK, DIM), lambda i: (i, 0)),
            pl.BlockSpec((DIM, DIM), lambda i: (0, 0)),
            pl.BlockSpec((DIM, 1), lambda i: (0, 0)),
        ],
        out_specs=pl.BlockSpec((_ROWS_PER_BLK, DIM), lambda i: (i, 0)),
        out_shape=jax.ShapeDtypeStruct((N_NODES, DIM), jnp.float32),
    )(x, w, b.reshape(DIM, 1))


def _tc_transform_mid(p, w, b):
    return pl.pallas_call(
        _transform_mid_body,
        grid=(_N_BLKS,),
        in_specs=[
            pl.BlockSpec((2, _ROWS_PER_BLK, DIM), lambda i: (0, i, 0)),
            pl.BlockSpec((DIM, DIM), lambda i: (0, 0)),
            pl.BlockSpec((DIM, 1), lambda i: (0, 0)),
        ],
        out_specs=pl.BlockSpec((_ROWS_PER_BLK, DIM), lambda i: (i, 0)),
        out_shape=jax.ShapeDtypeStruct((N_NODES, DIM), jnp.float32),
    )(p, w, b.reshape(DIM, 1))


def _tc_classifier(p, wt, b):
    return pl.pallas_call(
        _classifier_body,
        grid=(_N_BLKS,),
        in_specs=[
            pl.BlockSpec((2, _ROWS_PER_BLK, DIM), lambda i: (0, i, 0)),
            pl.BlockSpec((DIM, NUM_CLASSES), lambda i: (0, 0)),
            pl.BlockSpec((1, NUM_CLASSES), lambda i: (0, 0)),
        ],
        out_specs=pl.BlockSpec((_ROWS_PER_BLK, NUM_CLASSES), lambda i: (i, 0)),
        out_shape=jax.ShapeDtypeStruct((N_NODES, NUM_CLASSES), jnp.float32),
    )(p, wt, b.reshape(1, NUM_CLASSES))


def _sc_segment_sum(z, edge_index, zeros):
    mesh = plsc.VectorSubcoreMesh(core_axis_name="c", subcore_axis_name="s")

    @functools.partial(
        pl.kernel,
        mesh=mesh,
        out_type=jax.ShapeDtypeStruct((2, N_NODES, DIM), jnp.float32),
        scratch_types=[
            pltpu.VMEM((_SUPER,), jnp.int32),
            pltpu.VMEM((_SUPER,), jnp.int32),
            pltpu.VMEM((_SUPER,), jnp.int32),
            pltpu.VMEM((_CHUNK,), jnp.int32),
            pltpu.VMEM((_CHUNK,), jnp.int32),
            pltpu.VMEM((_CHUNK,), jnp.int32),
            pltpu.VMEM((_SUPER, DIM), jnp.float32),
            pltpu.VMEM((_SUPER, DIM), jnp.float32),
            pltpu.VMEM((_SUPER, DIM), jnp.float32),
            pltpu.VMEM_SHARED((N_NODES, DIM), jnp.float32),
            pltpu.SemaphoreType.DMA,
            pltpu.SemaphoreType.DMA,
            pltpu.SemaphoreType.DMA,
            pltpu.SemaphoreType.DMA,
            pltpu.SemaphoreType.DMA,
            pltpu.SemaphoreType.DMA,
            pltpu.SemaphoreType.DMA,
            pltpu.SemaphoreType.DMA,
            pltpu.SemaphoreType.DMA,
        ],
    )
    def seg(z_hbm, ei_hbm, zeros_hbm, out_hbm,
            sb0, sb1, sb2, d00, d10, d20, r0, r1, r2, acc,
            si0, si1, si2, sg0, sg1, sg2, ss0, ss1, ss2):
        sb = (sb0, sb1, sb2)
        db = ((d00,), (d10,), (d20,))
        rows = (r0, r1, r2)
        sem_i = (si0, si1, si2)
        sem_g = (sg0, sg1, sg2)
        sem_s = (ss0, ss1, ss2)

        cid = lax.axis_index("c")
        sid = lax.axis_index("s")
        wid = sid * 2 + cid
        rbase = pl.multiple_of(sid * _STRIPE + jnp.minimum(sid, 2) * _STRIPE_EXTRA, 8)

        def stripe_copy(src_ref, dst_ref):
            pltpu.sync_copy(src_ref.at[pl.ds(rbase, _STRIPE)],
                            dst_ref.at[pl.ds(rbase, _STRIPE)])

            @pl.when(sid < 2)
            def _():
                tail = pl.multiple_of(rbase + _STRIPE, 8)
                pltpu.sync_copy(src_ref.at[pl.ds(tail, _STRIPE_EXTRA)],
                                dst_ref.at[pl.ds(tail, _STRIPE_EXTRA)])

        stripe_copy(zeros_hbm, acc)
        plsc.subcore_barrier()

        def idx_start(b, s):
            off = pl.multiple_of(s * _SUPER, 8)
            pltpu.async_copy(ei_hbm.at[pl.ds(off, _SUPER)], sb[b], sem_i[b])
            for j in range(_K):
                pltpu.async_copy(
                    ei_hbm.at[pl.ds(N_EDGES + off + j * _CHUNK, _CHUNK)],
                    db[b][j], sem_i[b])

        def idx_wait(b):
            pltpu.make_async_copy(ei_hbm.at[pl.ds(0, _SUPER)], sb[b],
                                  sem_i[b]).wait()
            for j in range(_K):
                pltpu.make_async_copy(ei_hbm.at[pl.ds(0, _CHUNK)], db[b][j],
                                      sem_i[b]).wait()

        def gathers_start(b):
            for j in range(_K):
                pltpu.async_copy(z_hbm.at[sb[b].at[pl.ds(j * _CHUNK, _CHUNK)]],
                                 rows[b].at[pl.ds(j * _CHUNK, _CHUNK)],
                                 sem_g[b])

        def gathers_wait(b):
            for j in range(_K):
                pltpu.make_async_copy(
                    z_hbm.at[sb[b].at[pl.ds(j * _CHUNK, _CHUNK)]],
                    rows[b].at[pl.ds(j * _CHUNK, _CHUNK)], sem_g[b]).wait()

        def scatters_start(b):
            for j in range(_K):
                pltpu.async_copy(rows[b].at[pl.ds(j * _CHUNK, _CHUNK)],
                                 acc.at[db[b][j]], sem_s[b], add=True)

        def scatters_wait(b):
            for j in range(_K):
                pltpu.make_async_copy(rows[b].at[pl.ds(j * _CHUNK, _CHUNK)],
                                      acc.at[db[b][j]], sem_s[b]).wait()

        base_sup = wid * _NSUP
        idx_start(0, base_sup)

        def triple(t, carry):
            s0 = base_sup + 3 * t
            for b in range(3):
                bn = (b + 1) % 3
                bp = (b + 2) % 3
                idx_wait(b)
                gathers_start(b)
                if b == 2:
                    scatters_wait(bn)
                else:
                    @pl.when(t > 0)
                    def _(bn=bn):
                        scatters_wait(bn)
                if b == 2:
                    @pl.when(t < _NTRIP - 1)
                    def _():
                        idx_start(0, s0 + 3)
                else:
                    idx_start(bn, s0 + b + 1)
                if b == 0:
                    @pl.when(t > 0)
                    def _():
                        gathers_wait(2)
                        scatters_start(2)
                else:
                    gathers_wait(bp)
                    scatters_start(bp)
            return carry

        lax.fori_loop(0, _NTRIP, triple, 0)
        gathers_wait(2)
        scatters_start(2)
        scatters_wait(1)
        scatters_wait(2)

        @pl.when(wid < _EXTRA)
        def _():
            off = pl.multiple_of((_NW * _FULL + wid) * _CHUNK, 8)
            pltpu.sync_copy(ei_hbm.at[pl.ds(off, _CHUNK)],
                            sb0.at[pl.ds(0, _CHUNK)])
            pltpu.sync_copy(ei_hbm.at[pl.ds(N_EDGES + off, _CHUNK)], d00)
            pltpu.async_copy(z_hbm.at[sb0.at[pl.ds(0, _CHUNK)]],
                             r0.at[pl.ds(0, _CHUNK)], sg0).wait()
            pltpu.sync_copy(r0.at[pl.ds(0, _CHUNK)], acc.at[d00], add=True)

        plsc.subcore_barrier()
        stripe_copy(acc, out_hbm.at[cid])

    return seg(z, edge_index.reshape(2 * N_EDGES), zeros)


def kernel(edge_index, entity_embeddings, W1, b1, W2, b2, Wc, bc):
    zeros = jnp.zeros((N_NODES, DIM), jnp.float32)
    z1 = _tc_transform(entity_embeddings, W1, b1)
    p1 = _sc_segment_sum(z1, edge_index, zeros)
    z2 = _tc_transform_mid(p1, W2, b2)
    p2 = _sc_segment_sum(z2, edge_index, zeros)
    return _tc_classifier(p2, Wc.T, bc)


# speedup vs baseline: 13.0449x; 1.0158x over previous
import functools

import jax
import jax.numpy as jnp
from jax import lax
from jax.experimental import pallas as pl
from jax.experimental.pallas import tpu as pltpu
from jax.experimental.pallas import tpu_sc as plsc

N_NODES = 10000
N_EDGES = 320000
DIM = 128
NUM_CLASSES = 10
EPS = 1e-15

_ROWS_PER_BLK = 1000
_N_BLKS = N_NODES // _ROWS_PER_BLK

_CHUNK = 128
_K = 1
_SUPER = _K * _CHUNK
_N_CHUNKS = N_EDGES // _CHUNK
_NW = 32
_FULL = _N_CHUNKS // _NW
_EXTRA = _N_CHUNKS - _FULL * _NW
_NSUP = _FULL // _K
_NTRIP = _NSUP // 3
_STRIPE = 624
_STRIPE_EXTRA = 8


def _logmap0(x):
    sq = jnp.sum(x * x, axis=-1, keepdims=True)
    norm = jnp.maximum(jnp.sqrt(sq), EPS)
    arg = jnp.clip(norm, 0.0, 1.0 - 1e-6)
    att = 0.5 * jnp.log((1.0 + arg) / (1.0 - arg))
    return x * (att / norm)


def _expmap0(u):
    sq = jnp.sum(u * u, axis=-1, keepdims=True)
    norm = jnp.maximum(jnp.sqrt(sq), EPS)
    return jnp.tanh(norm) * u / norm


def _logmap0_scale_t(xt):
    sq = jnp.sum(xt * xt, axis=0, keepdims=True)
    inv = lax.rsqrt(jnp.maximum(sq, EPS * EPS))
    norm = sq * inv
    arg = jnp.clip(norm, 0.0, 1.0 - 1e-6)
    att = 0.5 * jnp.log((1.0 + arg) / (1.0 - arg))
    return att * inv


def _expmap0_scale_t(ut):
    sq = jnp.sum(ut * ut, axis=0, keepdims=True)
    inv = lax.rsqrt(jnp.maximum(sq, EPS * EPS))
    return jnp.tanh(sq * inv) * inv


def _transform_t(xt, w_ref, bc_ref):
    t = xt * _logmap0_scale_t(xt)
    yt = lax.dot(w_ref[...], t, preferred_element_type=jnp.float32) + bc_ref[...]
    return yt * _expmap0_scale_t(yt)


def _transform_body(x_ref, w_ref, bc_ref, z_ref):
    xt = x_ref[...].T
    z_ref[...] = _transform_t(xt, w_ref, bc_ref).T


def _transform_mid_body(p_ref, w_ref, bc_ref, z_ref):
    x = jnp.maximum(p_ref[0] + p_ref[1], 0.0)
    z_ref[...] = _transform_t(x.T, w_ref, bc_ref).T


def _classifier_body(p_ref, wt_ref, b_ref, o_ref):
    x = jnp.maximum(p_ref[0] + p_ref[1], 0.0)
    xt = x.T
    t = (xt * _logmap0_scale_t(xt)).T
    o_ref[...] = lax.dot(t, wt_ref[...], preferred_element_type=jnp.float32) + b_ref[...]


def _tc_transform(x, w, b):
    return pl.pallas_call(
        _transform_body,
        grid=(_N_BLKS,),
        in_specs=[
            pl.BlockSpec((_ROWS_PER_BLK, DIM), lambda i: (i, 0)),
            pl.BlockSpec((DIM, DIM), lambda i: (0, 0)),
            pl.BlockSpec((DIM, 1), lambda i: (0, 0)),
        ],
        out_specs=pl.BlockSpec((_ROWS_PER_BLK, DIM), lambda i: (i, 0)),
        out_shape=jax.ShapeDtypeStruct((N_NODES, DIM), jnp.float32),
    )(x, w, b.reshape(DIM, 1))


def _tc_transform_mid(p, w, b):
    return pl.pallas_call(
        _transform_mid_body,
        grid=(_N_BLKS,),
        in_specs=[
            pl.BlockSpec((2, _ROWS_PER_BLK, DIM), lambda i: (0, i, 0)),
            pl.BlockSpec((DIM, DIM), lambda i: (0, 0)),
            pl.BlockSpec((DIM, 1), lambda i: (0, 0)),
        ],
        out_specs=pl.BlockSpec((_ROWS_PER_BLK, DIM), lambda i: (i, 0)),
        out_shape=jax.ShapeDtypeStruct((N_NODES, DIM), jnp.float32),
    )(p, w, b.reshape(DIM, 1))


def _tc_classifier(p, wt, b):
    return pl.pallas_call(
        _classifier_body,
        grid=(_N_BLKS,),
        in_specs=[
            pl.BlockSpec((2, _ROWS_PER_BLK, DIM), lambda i: (0, i, 0)),
            pl.BlockSpec((DIM, NUM_CLASSES), lambda i: (0, 0)),
            pl.BlockSpec((1, NUM_CLASSES), lambda i: (0, 0)),
        ],
        out_specs=pl.BlockSpec((_ROWS_PER_BLK, NUM_CLASSES), lambda i: (i, 0)),
        out_shape=jax.ShapeDtypeStruct((N_NODES, NUM_CLASSES), jnp.float32),
    )(p, wt, b.reshape(1, NUM_CLASSES))


def _sc_segment_sum(z, edge_index, zeros):
    mesh = plsc.VectorSubcoreMesh(core_axis_name="c", subcore_axis_name="s")

    @functools.partial(
        pl.kernel,
        mesh=mesh,
        out_type=jax.ShapeDtypeStruct((2, N_NODES, DIM), jnp.float32),
        scratch_types=[
            pltpu.VMEM((2, _CHUNK), jnp.int32),
            pltpu.VMEM((2, _CHUNK), jnp.int32),
            pltpu.VMEM((2, _CHUNK), jnp.int32),
            pltpu.VMEM((_SUPER, DIM), jnp.float32),
            pltpu.VMEM((_SUPER, DIM), jnp.float32),
            pltpu.VMEM((_SUPER, DIM), jnp.float32),
            pltpu.VMEM_SHARED((N_NODES, DIM), jnp.float32),
            pltpu.SemaphoreType.DMA,
            pltpu.SemaphoreType.DMA,
            pltpu.SemaphoreType.DMA,
            pltpu.SemaphoreType.DMA,
            pltpu.SemaphoreType.DMA,
            pltpu.SemaphoreType.DMA,
            pltpu.SemaphoreType.DMA,
            pltpu.SemaphoreType.DMA,
            pltpu.SemaphoreType.DMA,
        ],
    )
    def seg(z_hbm, ei_hbm, zeros_hbm, out_hbm,
            eb0, eb1, eb2, r0, r1, r2, acc,
            si0, si1, si2, sg0, sg1, sg2, ss0, ss1, ss2):
        eb = (eb0, eb1, eb2)
        rows = (r0, r1, r2)
        sem_i = (si0, si1, si2)
        sem_g = (sg0, sg1, sg2)
        sem_s = (ss0, ss1, ss2)

        cid = lax.axis_index("c")
        sid = lax.axis_index("s")
        wid = sid * 2 + cid
        rbase = pl.multiple_of(sid * _STRIPE + jnp.minimum(sid, 2) * _STRIPE_EXTRA, 8)

        def stripe_copy(src_ref, dst_ref):
            pltpu.sync_copy(src_ref.at[pl.ds(rbase, _STRIPE)],
                            dst_ref.at[pl.ds(rbase, _STRIPE)])

            @pl.when(sid < 2)
            def _():
                tail = pl.multiple_of(rbase + _STRIPE, 8)
                pltpu.sync_copy(src_ref.at[pl.ds(tail, _STRIPE_EXTRA)],
                                dst_ref.at[pl.ds(tail, _STRIPE_EXTRA)])

        stripe_copy(zeros_hbm, acc)
        plsc.subcore_barrier()

        def idx_start(b, s):
            off = pl.multiple_of(s * _CHUNK, _CHUNK)
            pltpu.async_copy(ei_hbm.at[:, pl.ds(off, _CHUNK)], eb[b], sem_i[b])

        def idx_wait(b):
            pltpu.make_async_copy(ei_hbm.at[:, pl.ds(0, _CHUNK)], eb[b],
                                  sem_i[b]).wait()

        def gathers_start(b):
            pltpu.async_copy(z_hbm.at[eb[b].at[0]], rows[b], sem_g[b])

        def gathers_wait(b):
            pltpu.make_async_copy(z_hbm.at[eb[b].at[0]], rows[b],
                                  sem_g[b]).wait()

        def scatters_start(b):
            pltpu.async_copy(rows[b], acc.at[eb[b].at[1]], sem_s[b], add=True)

        def scatters_wait(b):
            pltpu.make_async_copy(rows[b], acc.at[eb[b].at[1]],
                                  sem_s[b]).wait()

        base_sup = wid * _NSUP
        idx_start(0, base_sup)

        def triple(t, carry):
            s0 = base_sup + 3 * t
            for b in range(3):
                bn = (b + 1) % 3
                bp = (b + 2) % 3
                idx_wait(b)
                gathers_start(b)
                if b == 2:
                    scatters_wait(bn)
                else:
                    @pl.when(t > 0)
                    def _(bn=bn):
                        scatters_wait(bn)
                if b == 2:
                    @pl.when(t < _NTRIP - 1)
                    def _():
                        idx_start(0, s0 + 3)
                else:
                    idx_start(bn, s0 + b + 1)
                if b == 0:
                    @pl.when(t > 0)
                    def _():
                        gathers_wait(2)
                        scatters_start(2)
                else:
                    gathers_wait(bp)
                    scatters_start(bp)
            return carry

        lax.fori_loop(0, _NTRIP, triple, 0)
        gathers_wait(2)
        scatters_start(2)
        scatters_wait(1)
        scatters_wait(2)

        @pl.when(wid < _EXTRA)
        def _():
            off = pl.multiple_of((_NW * _FULL + wid) * _CHUNK, _CHUNK)
            pltpu.sync_copy(ei_hbm.at[:, pl.ds(off, _CHUNK)], eb0)
            pltpu.async_copy(z_hbm.at[eb0.at[0]], r0, sg0).wait()
            pltpu.sync_copy(r0, acc.at[eb0.at[1]], add=True)

        plsc.subcore_barrier()
        stripe_copy(acc, out_hbm.at[cid])

    return seg(z, edge_index, zeros)


def kernel(edge_index, entity_embeddings, W1, b1, W2, b2, Wc, bc):
    zeros = jnp.zeros((N_NODES, DIM), jnp.float32)
    z1 = _tc_transform(entity_embeddings, W1, b1)
    p1 = _sc_segment_sum(z1, edge_index, zeros)
    z2 = _tc_transform_mid(p1, W2, b2)
    p2 = _sc_segment_sum(z2, edge_index, zeros)
    return _tc_classifier(p2, Wc.T, bc)


# speedup vs baseline: 13.0997x; 1.0042x over previous
import functools

import jax
import jax.numpy as jnp
from jax import lax
from jax.experimental import pallas as pl
from jax.experimental.pallas import tpu as pltpu
from jax.experimental.pallas import tpu_sc as plsc

N_NODES = 10000
N_EDGES = 320000
DIM = 128
NUM_CLASSES = 10
EPS = 1e-15

_ROWS_PER_BLK = 1000
_N_BLKS = N_NODES // _ROWS_PER_BLK

_CHUNK = 128
_K = 1
_SUPER = _K * _CHUNK
_N_CHUNKS = N_EDGES // _CHUNK
_NW = 32
_FULL = _N_CHUNKS // _NW
_EXTRA = _N_CHUNKS - _FULL * _NW
_NSUP = _FULL // _K
_NTRIP = _NSUP // 3
_STRIPE = 624
_STRIPE_EXTRA = 8


def _logmap0(x):
    sq = jnp.sum(x * x, axis=-1, keepdims=True)
    norm = jnp.maximum(jnp.sqrt(sq), EPS)
    arg = jnp.clip(norm, 0.0, 1.0 - 1e-6)
    att = 0.5 * jnp.log((1.0 + arg) / (1.0 - arg))
    return x * (att / norm)


def _expmap0(u):
    sq = jnp.sum(u * u, axis=-1, keepdims=True)
    norm = jnp.maximum(jnp.sqrt(sq), EPS)
    return jnp.tanh(norm) * u / norm


def _logmap0_scale_t(xt):
    sq = jnp.sum(xt * xt, axis=0, keepdims=True)
    inv = lax.rsqrt(jnp.maximum(sq, EPS * EPS))
    norm = sq * inv
    arg = jnp.clip(norm, 0.0, 1.0 - 1e-6)
    att = 0.5 * jnp.log((1.0 + arg) / (1.0 - arg))
    return att * inv


def _expmap0_scale_t(ut):
    sq = jnp.sum(ut * ut, axis=0, keepdims=True)
    inv = lax.rsqrt(jnp.maximum(sq, EPS * EPS))
    return jnp.tanh(sq * inv) * inv


def _transform_t(xt, w_ref, bc_ref):
    t = xt * _logmap0_scale_t(xt)
    yt = lax.dot(w_ref[...], t, preferred_element_type=jnp.float32) + bc_ref[...]
    return yt * _expmap0_scale_t(yt)


def _transform_body(x_ref, w_ref, bc_ref, z_ref):
    xt = x_ref[...].T
    z_ref[...] = _transform_t(xt, w_ref, bc_ref).T


def _transform_mid_body(p_ref, w_ref, bc_ref, z_ref):
    x = jnp.maximum(p_ref[0] + p_ref[1], 0.0)
    z_ref[...] = _transform_t(x.T, w_ref, bc_ref).T


def _classifier_body(p_ref, wt_ref, b_ref, o_ref):
    x = jnp.maximum(p_ref[0] + p_ref[1], 0.0)
    xt = x.T
    t = (xt * _logmap0_scale_t(xt)).T
    o_ref[...] = lax.dot(t, wt_ref[...], preferred_element_type=jnp.float32) + b_ref[...]


def _tc_transform(x, w, b):
    return pl.pallas_call(
        _transform_body,
        grid=(_N_BLKS,),
        in_specs=[
            pl.BlockSpec((_ROWS_PER_BLK, DIM), lambda i: (i, 0)),
            pl.BlockSpec((DIM, DIM), lambda i: (0, 0)),
            pl.BlockSpec((DIM, 1), lambda i: (0, 0)),
        ],
        out_specs=pl.BlockSpec((_ROWS_PER_BLK, DIM), lambda i: (i, 0)),
        out_shape=jax.ShapeDtypeStruct((N_NODES, DIM), jnp.float32),
    )(x, w, b.reshape(DIM, 1))


def _tc_transform_mid(p, w, b):
    return pl.pallas_call(
        _transform_mid_body,
        grid=(_N_BLKS,),
        in_specs=[
            pl.BlockSpec((2, _ROWS_PER_BLK, DIM), lambda i: (0, i, 0)),
            pl.BlockSpec((DIM, DIM), lambda i: (0, 0)),
            pl.BlockSpec((DIM, 1), lambda i: (0, 0)),
        ],
        out_specs=pl.BlockSpec((_ROWS_PER_BLK, DIM), lambda i: (i, 0)),
        out_shape=jax.ShapeDtypeStruct((N_NODES, DIM), jnp.float32),
    )(p, w, b.reshape(DIM, 1))


def _tc_classifier(p, wt, b):
    return pl.pallas_call(
        _classifier_body,
        grid=(_N_BLKS,),
        in_specs=[
            pl.BlockSpec((2, _ROWS_PER_BLK, DIM), lambda i: (0, i, 0)),
            pl.BlockSpec((DIM, NUM_CLASSES), lambda i: (0, 0)),
            pl.BlockSpec((1, NUM_CLASSES), lambda i: (0, 0)),
        ],
        out_specs=pl.BlockSpec((_ROWS_PER_BLK, NUM_CLASSES), lambda i: (i, 0)),
        out_shape=jax.ShapeDtypeStruct((N_NODES, NUM_CLASSES), jnp.float32),
    )(p, wt, b.reshape(1, NUM_CLASSES))


def _sc_segment_sum(z, edge_index, zeros):
    mesh = plsc.VectorSubcoreMesh(core_axis_name="c", subcore_axis_name="s")

    @functools.partial(
        pl.kernel,
        mesh=mesh,
        out_type=jax.ShapeDtypeStruct((2, N_NODES, DIM), jnp.float32),
        scratch_types=[
            pltpu.VMEM((2, _CHUNK), jnp.int32),
            pltpu.VMEM((2, _CHUNK), jnp.int32),
            pltpu.VMEM((2, _CHUNK), jnp.int32),
            pltpu.VMEM((_SUPER, DIM), jnp.float32),
            pltpu.VMEM((_SUPER, DIM), jnp.float32),
            pltpu.VMEM((_SUPER, DIM), jnp.float32),
            pltpu.VMEM_SHARED((N_NODES, DIM), jnp.float32),
            pltpu.SemaphoreType.DMA,
            pltpu.SemaphoreType.DMA,
            pltpu.SemaphoreType.DMA,
            pltpu.SemaphoreType.DMA,
            pltpu.SemaphoreType.DMA,
            pltpu.SemaphoreType.DMA,
            pltpu.SemaphoreType.DMA,
            pltpu.SemaphoreType.DMA,
            pltpu.SemaphoreType.DMA,
        ],
    )
    def seg(z_hbm, ei_hbm, zeros_hbm, out_hbm,
            eb0, eb1, eb2, r0, r1, r2, acc,
            si0, si1, si2, sg0, sg1, sg2, ss0, ss1, ss2):
        eb = (eb0, eb1, eb2)
        rows = (r0, r1, r2)
        sem_i = (si0, si1, si2)
        sem_g = (sg0, sg1, sg2)
        sem_s = (ss0, ss1, ss2)

        cid = lax.axis_index("c")
        sid = lax.axis_index("s")
        wid = sid * 2 + cid
        rbase = pl.multiple_of(sid * _STRIPE + jnp.minimum(sid, 2) * _STRIPE_EXTRA, 8)

        def stripe_copy(src_ref, dst_ref):
            pltpu.sync_copy(src_ref.at[pl.ds(rbase, _STRIPE)],
                            dst_ref.at[pl.ds(rbase, _STRIPE)])

            @pl.when(sid < 2)
            def _():
                tail = pl.multiple_of(rbase + _STRIPE, 8)
                pltpu.sync_copy(src_ref.at[pl.ds(tail, _STRIPE_EXTRA)],
                                dst_ref.at[pl.ds(tail, _STRIPE_EXTRA)])

        def idx_start(b, s):
            off = pl.multiple_of(s * _CHUNK, _CHUNK)
            pltpu.async_copy(ei_hbm.at[:, pl.ds(off, _CHUNK)], eb[b], sem_i[b])

        def idx_wait(b):
            pltpu.make_async_copy(ei_hbm.at[:, pl.ds(0, _CHUNK)], eb[b],
                                  sem_i[b]).wait()

        def gathers_start(b):
            pltpu.async_copy(z_hbm.at[eb[b].at[0]], rows[b], sem_g[b])

        def gathers_wait(b):
            pltpu.make_async_copy(z_hbm.at[eb[b].at[0]], rows[b],
                                  sem_g[b]).wait()

        def scatters_start(b):
            pltpu.async_copy(rows[b], acc.at[eb[b].at[1]], sem_s[b], add=True)

        def scatters_wait(b):
            pltpu.make_async_copy(rows[b], acc.at[eb[b].at[1]],
                                  sem_s[b]).wait()

        base_sup = wid * _NSUP
        idx_start(0, base_sup)
        stripe_copy(zeros_hbm, acc)
        plsc.subcore_barrier()

        def triple(t, carry):
            s0 = base_sup + 3 * t
            for b in range(3):
                bn = (b + 1) % 3
                bp = (b + 2) % 3
                idx_wait(b)
                gathers_start(b)
                if b == 2:
                    scatters_wait(bn)
                else:
                    @pl.when(t > 0)
                    def _(bn=bn):
                        scatters_wait(bn)
                if b == 2:
                    @pl.when(t < _NTRIP - 1)
                    def _():
                        idx_start(0, s0 + 3)
                else:
                    idx_start(bn, s0 + b + 1)
                if b == 0:
                    @pl.when(t > 0)
                    def _():
                        gathers_wait(2)
                        scatters_start(2)
                else:
                    gathers_wait(bp)
                    scatters_start(bp)
            return carry

        lax.fori_loop(0, _NTRIP, triple, 0)
        gathers_wait(2)
        scatters_start(2)
        scatters_wait(1)
        scatters_wait(2)

        @pl.when(wid < _EXTRA)
        def _():
            off = pl.multiple_of((_NW * _FULL + wid) * _CHUNK, _CHUNK)
            pltpu.sync_copy(ei_hbm.at[:, pl.ds(off, _CHUNK)], eb0)
            pltpu.async_copy(z_hbm.at[eb0.at[0]], r0, sg0).wait()
            pltpu.sync_copy(r0, acc.at[eb0.at[1]], add=True)

        plsc.subcore_barrier()
        stripe_copy(acc, out_hbm.at[cid])

    return seg(z, edge_index, zeros)


def kernel(edge_index, entity_embeddings, W1, b1, W2, b2, Wc, bc):
    zeros = jnp.zeros((N_NODES, DIM), jnp.float32)
    z1 = _tc_transform(entity_embeddings, W1, b1)
    p1 = _sc_segment_sum(z1, edge_index, zeros)
    z2 = _tc_transform_mid(p1, W2, b2)
    p2 = _sc_segment_sum(z2, edge_index, zeros)
    return _tc_classifier(p2, Wc.T, bc)
